# baseline (speedup 1.0000x reference)
import functools

import jax
import jax.numpy as jnp
from jax import lax
from jax.experimental import pallas as pl
from jax.experimental.pallas import tpu as pltpu
from jax.experimental.pallas import tpu_sc as plsc

_D = 64
_V = 6
_GRP = 16


@functools.cache
def _build(NB: int, S: int):
    info = plsc.get_sparse_core_info()
    nw = info.num_cores * info.num_subcores
    nb_per_w = NB // nw
    n_grp = -(-S // _GRP)
    last = S - _GRP
    assert NB % nw == 0 and nb_per_w % 2 == 0 and S % 8 == 0
    mesh = plsc.VectorSubcoreMesh(core_axis_name="c", subcore_axis_name="s")

    @functools.partial(
        pl.kernel,
        mesh=mesh,
        out_type=jax.ShapeDtypeStruct((NB, S, _D), jnp.float32),
        scratch_types=[
            pltpu.VMEM((_V * _D,), jnp.float32),
            pltpu.VMEM((nb_per_w, S), jnp.int32),
            pltpu.VMEM((2, S, _D), jnp.float32),
            pltpu.SemaphoreType.DMA,
            pltpu.SemaphoreType.DMA,
        ],
        compiler_params=pltpu.CompilerParams(needs_layout_passes=False),
    )
    def k(tflat_hbm, ids_hbm, out_hbm, tflat_v, ids_v, rows_v, o0, o1):
        osem = (o0, o1)
        wid = lax.axis_index("s") * info.num_cores + lax.axis_index("c")
        pltpu.sync_copy(tflat_hbm, tflat_v)
        pltpu.sync_copy(ids_hbm.at[pl.ds(wid * nb_per_w, nb_per_w)], ids_v)
        iota = lax.iota(jnp.int32, _GRP)

        def body(g, carry):
            for b in range(2):
                c = 2 * g + b
                out_slice = out_hbm.at[wid * nb_per_w + c]

                @pl.when(g > 0)
                def _wait_prev():
                    pltpu.make_async_copy(rows_v.at[b], out_slice,
                                          osem[b]).wait()

                def grp(i, carry2):
                    col = jnp.minimum(i * _GRP, last)
                    v_ids = ids_v[c, pl.ds(col, _GRP)]
                    v_off = v_ids * _D
                    for r in range(_GRP):
                        bc = lax.gather(
                            v_off,
                            jnp.full((_GRP, 1), r, jnp.int32),
                            lax.GatherDimensionNumbers(
                                offset_dims=(), collapsed_slice_dims=(0,),
                                start_index_map=(0,)),
                            (1,),
                            mode=lax.GatherScatterMode.PROMISE_IN_BOUNDS)
                        row = col + r
                        for j in range(_D // _GRP):
                            vals = plsc.load_gather(
                                tflat_v, [bc + (iota + j * _GRP)])
                            rows_v[b, row, pl.ds(j * _GRP, _GRP)] = vals
                    return carry2

                lax.fori_loop(0, n_grp, grp, 0)
                pltpu.make_async_copy(rows_v.at[b], out_slice,
                                      osem[b]).start()
            return carry

        lax.fori_loop(0, nb_per_w // 2, body, 0)
        for b in range(2):
            c = nb_per_w - 2 + b
            out_slice = out_hbm.at[wid * nb_per_w + c]
            pltpu.make_async_copy(rows_v.at[b], out_slice, osem[b]).wait()

    return k


def kernel(tone_ids, embed_weight):
    b, s = tone_ids.shape
    return _build(b, s)(embed_weight.reshape(-1), tone_ids.astype(jnp.int32))


# speedup vs baseline: 4.9269x; 4.9269x over previous
import functools

import jax
import jax.numpy as jnp
from jax import lax
from jax.experimental import pallas as pl
from jax.experimental.pallas import tpu as pltpu
from jax.experimental.pallas import tpu_sc as plsc

_D = 64
_V = 6
_GRP = 16
_SB = 8
_DB = 8
_BB = 256


def _bcast_gather(src, idx):
    return lax.gather(
        src, idx.reshape(_GRP, 1),
        lax.GatherDimensionNumbers(
            offset_dims=(), collapsed_slice_dims=(0,), start_index_map=(0,)),
        (1,),
        mode=lax.GatherScatterMode.PROMISE_IN_BOUNDS)


@functools.cache
def _build(NB: int, S: int):
    info = plsc.get_sparse_core_info()
    nw = info.num_cores * info.num_subcores
    n_units = (S // _SB) * (_D // _DB) * (NB // _BB)
    u_per_w = n_units // nw
    n_grp = _SB * (_BB // _GRP)
    assert n_units % nw == 0 and u_per_w % 2 == 0
    mesh = plsc.VectorSubcoreMesh(core_axis_name="c", subcore_axis_name="s")

    def decode(u):
        s_hi = u // ((_D // _DB) * (NB // _BB))
        rem = u % ((_D // _DB) * (NB // _BB))
        d_hi = rem // (NB // _BB)
        q = rem % (NB // _BB)
        return s_hi, d_hi, q

    @functools.partial(
        pl.kernel,
        mesh=mesh,
        out_type=jax.ShapeDtypeStruct((S, _D, NB), jnp.float32),
        scratch_types=[
            pltpu.VMEM((_D * _GRP,), jnp.float32),
            pltpu.VMEM((2, _SB, _BB), jnp.int32),
            pltpu.VMEM((2, _SB, _DB, _BB), jnp.float32),
            pltpu.SemaphoreType.DMA,
            pltpu.SemaphoreType.DMA,
            pltpu.SemaphoreType.DMA,
            pltpu.SemaphoreType.DMA,
        ],
        compiler_params=pltpu.CompilerParams(needs_layout_passes=False),
    )
    def k(tcm_hbm, ids_hbm, out_hbm, tcm_v, ids_v, rows_v, i0, i1, o0, o1):
        isem = (i0, i1)
        osem = (o0, o1)
        wid = lax.axis_index("s") * info.num_cores + lax.axis_index("c")
        pltpu.sync_copy(tcm_hbm, tcm_v)

        def ids_copy(u, b):
            s_hi, _, q = decode(u)
            return pltpu.make_async_copy(
                ids_hbm.at[pl.ds(s_hi * _SB, _SB), pl.ds(q * _BB, _BB)],
                ids_v.at[b], isem[b])

        def out_copy(u, b):
            s_hi, d_hi, q = decode(u)
            return pltpu.make_async_copy(
                rows_v.at[b],
                out_hbm.at[pl.ds(s_hi * _SB, _SB), pl.ds(d_hi * _DB, _DB),
                           pl.ds(q * _BB, _BB)], osem[b])

        ids_copy(wid, 0).start()

        def body(g, carry):
            for b in range(2):
                t = 2 * g + b
                u = wid + nw * t
                _, d_hi, _ = decode(u)

                @pl.when(t + 1 < u_per_w)
                def _prefetch():
                    ids_copy(wid + nw * (t + 1), 1 - b).start()

                ids_copy(u, b).wait()

                @pl.when(g > 0)
                def _wait_prev():
                    out_copy(wid + nw * (t - 2), b).wait()

                cols = [tcm_v[pl.ds((d_hi * _DB + dl) * _GRP, _GRP)]
                        for dl in range(_DB)]

                def grp(i, carry2):
                    s_lo = i // (_BB // _GRP)
                    gg = i % (_BB // _GRP)
                    v_ids = ids_v[b, s_lo, pl.ds(gg * _GRP, _GRP)]
                    for dl in range(_DB):
                        vals = _bcast_gather(cols[dl], v_ids)
                        rows_v[b, s_lo, dl, pl.ds(gg * _GRP, _GRP)] = vals
                    return carry2

                lax.fori_loop(0, n_grp, grp, 0)
                out_copy(u, b).start()
            return carry

        lax.fori_loop(0, u_per_w // 2, body, 0)
        for b in range(2):
            t = u_per_w - 2 + b
            out_copy(wid + nw * t, b).wait()

    return k


def kernel(tone_ids, embed_weight):
    b, s = tone_ids.shape
    tcm = jnp.zeros((_D, _GRP), jnp.float32)
    tcm = tcm.at[:, :_V].set(embed_weight.T).reshape(-1)
    out = _build(b, s)(tcm, tone_ids.T.astype(jnp.int32))
    return out.transpose(2, 0, 1)


# speedup vs baseline: 5.0592x; 1.0268x over previous
import functools

import jax
import jax.numpy as jnp
from jax import lax
from jax.experimental import pallas as pl
from jax.experimental.pallas import tpu as pltpu
from jax.experimental.pallas import tpu_sc as plsc

_D = 64
_V = 6
_GRP = 16
_SB = 4
_DB = 8
_BB = 1024


def _bcast_gather(src, idx):
    return lax.gather(
        src, idx.reshape(_GRP, 1),
        lax.GatherDimensionNumbers(
            offset_dims=(), collapsed_slice_dims=(0,), start_index_map=(0,)),
        (1,),
        mode=lax.GatherScatterMode.PROMISE_IN_BOUNDS)


@functools.cache
def _build(NB: int, S: int):
    info = plsc.get_sparse_core_info()
    nw = info.num_cores * info.num_subcores
    n_units = (S // _SB) * (_D // _DB) * (NB // _BB)
    u_per_w = n_units // nw
    n_grp = _SB * (_BB // _GRP)
    assert n_units % nw == 0 and u_per_w % 2 == 0
    mesh = plsc.VectorSubcoreMesh(core_axis_name="c", subcore_axis_name="s")

    def decode(u):
        s_hi = u // ((_D // _DB) * (NB // _BB))
        rem = u % ((_D // _DB) * (NB // _BB))
        d_hi = rem // (NB // _BB)
        q = rem % (NB // _BB)
        return s_hi, d_hi, q

    @functools.partial(
        pl.kernel,
        mesh=mesh,
        out_type=jax.ShapeDtypeStruct((S, _D, NB), jnp.float32),
        scratch_types=[
            pltpu.VMEM((_D * _GRP,), jnp.float32),
            pltpu.VMEM((2, _SB, _BB), jnp.int32),
            pltpu.VMEM((2, _SB, _DB, _BB), jnp.float32),
            pltpu.SemaphoreType.DMA,
            pltpu.SemaphoreType.DMA,
            pltpu.SemaphoreType.DMA,
            pltpu.SemaphoreType.DMA,
        ],
        compiler_params=pltpu.CompilerParams(needs_layout_passes=False),
    )
    def k(tcm_hbm, ids_hbm, out_hbm, tcm_v, ids_v, rows_v, i0, i1, o0, o1):
        isem = (i0, i1)
        osem = (o0, o1)
        wid = lax.axis_index("s") * info.num_cores + lax.axis_index("c")
        pltpu.sync_copy(tcm_hbm, tcm_v)

        def ids_copy(u, b):
            s_hi, _, q = decode(u)
            return pltpu.make_async_copy(
                ids_hbm.at[pl.ds(s_hi * _SB, _SB), pl.ds(q * _BB, _BB)],
                ids_v.at[b], isem[b])

        def out_copy(u, b):
            s_hi, d_hi, q = decode(u)
            return pltpu.make_async_copy(
                rows_v.at[b],
                out_hbm.at[pl.ds(s_hi * _SB, _SB), pl.ds(d_hi * _DB, _DB),
                           pl.ds(q * _BB, _BB)], osem[b])

        ids_copy(wid, 0).start()

        def body(g, carry):
            for b in range(2):
                t = 2 * g + b
                u = wid + nw * t
                _, d_hi, _ = decode(u)

                @pl.when(t + 1 < u_per_w)
                def _prefetch():
                    ids_copy(wid + nw * (t + 1), 1 - b).start()

                ids_copy(u, b).wait()

                @pl.when(g > 0)
                def _wait_prev():
                    out_copy(wid + nw * (t - 2), b).wait()

                cols = [tcm_v[pl.ds((d_hi * _DB + dl) * _GRP, _GRP)]
                        for dl in range(_DB)]

                def grp(i, carry2):
                    s_lo = i // (_BB // _GRP)
                    gg = i % (_BB // _GRP)
                    v_ids = ids_v[b, s_lo, pl.ds(gg * _GRP, _GRP)]
                    for dl in range(_DB):
                        vals = _bcast_gather(cols[dl], v_ids)
                        rows_v[b, s_lo, dl, pl.ds(gg * _GRP, _GRP)] = vals
                    return carry2

                lax.fori_loop(0, n_grp, grp, 0)
                out_copy(u, b).start()
            return carry

        lax.fori_loop(0, u_per_w // 2, body, 0)
        for b in range(2):
            t = u_per_w - 2 + b
            out_copy(wid + nw * t, b).wait()

    return k


def kernel(tone_ids, embed_weight):
    b, s = tone_ids.shape
    tcm = jnp.zeros((_D, _GRP), jnp.float32)
    tcm = tcm.at[:, :_V].set(embed_weight.T).reshape(-1)
    out = _build(b, s)(tcm, tone_ids.T.astype(jnp.int32))
    return out.transpose(2, 0, 1)


# speedup vs baseline: 5.0610x; 1.0004x over previous
import functools

import jax
import jax.numpy as jnp
from jax import lax
from jax.experimental import pallas as pl
from jax.experimental.pallas import tpu as pltpu
from jax.experimental.pallas import tpu_sc as plsc

_D = 64
_V = 6
_GRP = 16
_SB = 1
_DB = 8
_BB = 4096


def _bcast_gather(src, idx):
    return lax.gather(
        src, idx.reshape(_GRP, 1),
        lax.GatherDimensionNumbers(
            offset_dims=(), collapsed_slice_dims=(0,), start_index_map=(0,)),
        (1,),
        mode=lax.GatherScatterMode.PROMISE_IN_BOUNDS)


@functools.cache
def _build(NB: int, S: int):
    info = plsc.get_sparse_core_info()
    nw = info.num_cores * info.num_subcores
    n_units = (S // _SB) * (_D // _DB) * (NB // _BB)
    u_per_w = n_units // nw
    n_grp = _SB * (_BB // _GRP)
    assert n_units % nw == 0 and u_per_w % 2 == 0
    mesh = plsc.VectorSubcoreMesh(core_axis_name="c", subcore_axis_name="s")

    def decode(u):
        s_hi = u // ((_D // _DB) * (NB // _BB))
        rem = u % ((_D // _DB) * (NB // _BB))
        d_hi = rem // (NB // _BB)
        q = rem % (NB // _BB)
        return s_hi, d_hi, q

    @functools.partial(
        pl.kernel,
        mesh=mesh,
        out_type=jax.ShapeDtypeStruct((S, _D, NB), jnp.float32),
        scratch_types=[
            pltpu.VMEM((_D * _GRP,), jnp.float32),
            pltpu.VMEM((2, _SB, _BB), jnp.int32),
            pltpu.VMEM((2, _SB, _DB, _BB), jnp.float32),
            pltpu.SemaphoreType.DMA,
            pltpu.SemaphoreType.DMA,
            pltpu.SemaphoreType.DMA,
            pltpu.SemaphoreType.DMA,
        ],
        compiler_params=pltpu.CompilerParams(needs_layout_passes=False),
    )
    def k(tcm_hbm, ids_hbm, out_hbm, tcm_v, ids_v, rows_v, i0, i1, o0, o1):
        isem = (i0, i1)
        osem = (o0, o1)
        wid = lax.axis_index("s") * info.num_cores + lax.axis_index("c")
        pltpu.sync_copy(tcm_hbm, tcm_v)

        def ids_copy(u, b):
            s_hi, _, q = decode(u)
            return pltpu.make_async_copy(
                ids_hbm.at[pl.ds(s_hi * _SB, _SB), pl.ds(q * _BB, _BB)],
                ids_v.at[b], isem[b])

        def out_copy(u, b):
            s_hi, d_hi, q = decode(u)
            return pltpu.make_async_copy(
                rows_v.at[b],
                out_hbm.at[pl.ds(s_hi * _SB, _SB), pl.ds(d_hi * _DB, _DB),
                           pl.ds(q * _BB, _BB)], osem[b])

        ids_copy(wid, 0).start()

        def body(g, carry):
            for b in range(2):
                t = 2 * g + b
                u = wid + nw * t
                _, d_hi, _ = decode(u)

                @pl.when(t + 1 < u_per_w)
                def _prefetch():
                    ids_copy(wid + nw * (t + 1), 1 - b).start()

                ids_copy(u, b).wait()

                @pl.when(g > 0)
                def _wait_prev():
                    out_copy(wid + nw * (t - 2), b).wait()

                cols = [tcm_v[pl.ds((d_hi * _DB + dl) * _GRP, _GRP)]
                        for dl in range(_DB)]

                def grp(i, carry2):
                    s_lo = i // (_BB // _GRP)
                    gg = i % (_BB // _GRP)
                    v_ids = ids_v[b, s_lo, pl.ds(gg * _GRP, _GRP)]
                    for dl in range(_DB):
                        vals = _bcast_gather(cols[dl], v_ids)
                        rows_v[b, s_lo, dl, pl.ds(gg * _GRP, _GRP)] = vals
                    return carry2

                lax.fori_loop(0, n_grp, grp, 0)
                out_copy(u, b).start()
            return carry

        lax.fori_loop(0, u_per_w // 2, body, 0)
        for b in range(2):
            t = u_per_w - 2 + b
            out_copy(wid + nw * t, b).wait()

    return k


def kernel(tone_ids, embed_weight):
    b, s = tone_ids.shape
    tcm = jnp.zeros((_D, _GRP), jnp.float32)
    tcm = tcm.at[:, :_V].set(embed_weight.T).reshape(-1)
    out = _build(b, s)(tcm, tone_ids.T.astype(jnp.int32))
    return out.transpose(2, 0, 1)
